# Initial kernel scaffold; baseline (speedup 1.0000x reference)
import jax
import jax.numpy as jnp
from jax.experimental import pallas as pl


def kernel(x, W, gamma, beta, k):
    raise NotImplementedError("write your pallas kernel here")


# speedup vs baseline: 3.4872x; 3.4872x over previous
import functools
import jax
import jax.numpy as jnp
from jax import lax
from jax.experimental import pallas as pl
from jax.experimental.pallas import tpu as pltpu
from jax.experimental.pallas import tpu_sc as plsc


B, P, Q, O = 1024, 32, 32, 32
BB = 32
NSTEP = B // BB
NW = 32
BPW = B // NW


def _pass1_body(x_ref, wd_ref, w2_ref, rec_ref, stats_ref):
    i = pl.program_id(0)
    X = x_ref[...]
    Wd = wd_ref[...]
    W2 = w2_ref[...]

    dn = (((2,), (1,)), ((), ()))
    U = jax.lax.dot_general(X, Wd, dn, preferred_element_type=jnp.float32)
    V = jax.lax.dot_general(X, W2, dn, preferred_element_type=jnp.float32)

    sq = jnp.sum(X * X, axis=1)
    G = jax.lax.dot_general(X, X, (((1,), (1,)), ((0,), (0,))),
                            preferred_element_type=jnp.float32)
    d2 = sq[:, :, None] + sq[:, None, :] - 2.0 * G
    d2 = jnp.maximum(d2, 0.0)

    a = d2[:, :, :, None]
    bb = d2[:, :, None, :]
    rp = jax.lax.broadcasted_iota(jnp.int32, (1, 1, Q, Q), 2)
    rr = jax.lax.broadcasted_iota(jnp.int32, (1, 1, Q, Q), 3)
    cmp = (a < bb) | ((a == bb) & (rp < rr))
    rank = jnp.sum(cmp.astype(jnp.float32), axis=2)

    jj = jax.lax.broadcasted_iota(jnp.int32, (1, 1, 1, Q), 3).astype(jnp.float32)
    oh = (rank[:, :, :, None] == jj).astype(jnp.float32)
    rf = jax.lax.broadcasted_iota(jnp.int32, (1, 1, Q, 1), 2).astype(jnp.float32)
    idxf = jnp.sum(oh * rf, axis=2)

    rec_ref[...] = jnp.concatenate([U, V, idxf], axis=1)

    ub = jnp.sum(U, axis=1)
    vb = jnp.sum(V, axis=1)
    su = jnp.sum(ub, axis=0)
    sv = jnp.sum(vb, axis=0)
    su2 = jnp.sum(U * U, axis=(0, 1))
    sv2 = jnp.sum(V * V, axis=(0, 1))
    sx = jnp.sum(ub * vb, axis=0)
    zero = jnp.zeros((O,), jnp.float32)
    new = jnp.stack([su, sv, su2, sv2, sx, zero, zero, zero], axis=0)

    @pl.when(i == 0)
    def _():
        stats_ref[...] = new

    @pl.when(i != 0)
    def _():
        stats_ref[...] = stats_ref[...] + new


def _run_pass1(x, Wd, W2):
    return pl.pallas_call(
        _pass1_body,
        grid=(NSTEP,),
        in_specs=[
            pl.BlockSpec((BB, P, Q), lambda i: (i, 0, 0)),
            pl.BlockSpec((O, Q), lambda i: (0, 0)),
            pl.BlockSpec((O, Q), lambda i: (0, 0)),
        ],
        out_specs=[
            pl.BlockSpec((BB, 3 * P, Q), lambda i: (i, 0, 0)),
            pl.BlockSpec((8, O), lambda i: (0, 0)),
        ],
        out_shape=[
            jax.ShapeDtypeStruct((B, 3 * P, Q), jnp.float32),
            jax.ShapeDtypeStruct((8, O), jnp.float32),
        ],
    )(x, Wd, W2)


def _lane_splat(vec, lane):
    idxs = jnp.full((16, 1), lane, jnp.int32)
    return jax.lax.gather(
        vec, idxs,
        jax.lax.GatherDimensionNumbers(
            offset_dims=(), collapsed_slice_dims=(0,), start_index_map=(0,)),
        slice_sizes=(1,),
        mode=jax.lax.GatherScatterMode.PROMISE_IN_BOUNDS)


def _pass2_body(rec_hbm, ss_hbm, out_hbm, rec_v, ss_v, us_v, vs_v, out_v):
    wid = lax.axis_index("s") * 2 + lax.axis_index("c")
    pltpu.sync_copy(ss_hbm, ss_v)

    def batch_body(t, carry):
        b = wid * BPW + t
        pltpu.sync_copy(rec_hbm.at[b], rec_v)

        for p in range(P):
            for h in range(2):
                sv = ss_v[pl.ds(16 * h, 16)]
                sh = ss_v[pl.ds(32 + 16 * h, 16)]
                us_v[pl.ds(p * 32 + 16 * h, 16)] = (
                    rec_v[p, pl.ds(16 * h, 16)] * sv + sh)
                vs_v[pl.ds(p * 32 + 16 * h, 16)] = (
                    rec_v[P + p, pl.ds(16 * h, 16)] * sv)

        def p_body(p, c):
            idx_lo = rec_v[2 * P + p, pl.ds(0, 16)].astype(jnp.int32)
            idx_hi = rec_v[2 * P + p, pl.ds(16, 16)].astype(jnp.int32)
            base_lo = idx_lo * 32
            base_hi = idx_hi * 32
            us_lo = us_v[pl.ds(p * 32, 16)]
            us_hi = us_v[pl.ds(p * 32 + 16, 16)]
            for o in range(O):
                usp = _lane_splat(us_lo if o < 16 else us_hi, o % 16)
                vlo = plsc.load_gather(vs_v, [base_lo + o])
                vhi = plsc.load_gather(vs_v, [base_hi + o])
                out_v[pl.ds(o * 1024 + p * 32, 16)] = jnp.maximum(vlo + usp, 0.0)
                out_v[pl.ds(o * 1024 + p * 32 + 16, 16)] = jnp.maximum(vhi + usp, 0.0)
            return c

        lax.fori_loop(0, P, p_body, 0, unroll=False)
        pltpu.sync_copy(out_v, out_hbm.at[b])
        return carry

    lax.fori_loop(0, BPW, batch_body, 0, unroll=False)


def _run_pass2(rec, ss):
    mesh = plsc.VectorSubcoreMesh(core_axis_name="c", subcore_axis_name="s")
    f = pl.kernel(
        _pass2_body,
        out_type=jax.ShapeDtypeStruct((B, O * P * Q), jnp.float32),
        mesh=mesh,
        compiler_params=pltpu.CompilerParams(needs_layout_passes=False),
        scratch_types=[
            pltpu.VMEM((3 * P, Q), jnp.float32),
            pltpu.VMEM((64,), jnp.float32),
            pltpu.VMEM((P * O,), jnp.float32),
            pltpu.VMEM((P * O,), jnp.float32),
            pltpu.VMEM((O * P * Q,), jnp.float32),
        ],
    )
    return f(rec, ss)


def kernel(x, W, gamma, beta, k):
    W2d = W[:, :, 0, 0]
    W1 = W2d[:, :Q]
    W2 = W2d[:, Q:]
    rec, stats = _run_pass1(x, W1 - W2, W2)

    su, sv, su2, sv2, sx = stats[0], stats[1], stats[2], stats[3], stats[4]
    kf = jnp.asarray(k, jnp.float32)
    n = B * P * kf
    mean = (kf * su + P * sv) / n
    e2 = (kf * su2 + 2.0 * sx + P * sv2) / n
    var = e2 - mean * mean
    s = gamma * jax.lax.rsqrt(var + 1e-5)
    shift = beta - mean * s
    ss = jnp.concatenate([s, shift], axis=0)

    out = _run_pass2(rec, ss)
    return out.reshape(B, O, P, Q)
